# baseline (speedup 1.0000x reference)
import functools

import jax
import jax.numpy as jnp
from jax import lax
from jax.experimental import pallas as pl
from jax.experimental.pallas import tpu as pltpu
from jax.experimental.pallas import tpu_sc as plsc

_LANES = 128


def _table_body(w_ref, b_ref, out_ref):
    out_ref[...] = w_ref[...].T + b_ref[...]


def _build_table(w_pad, b_row):
    e, cp = w_pad.shape
    return pl.pallas_call(
        _table_body,
        out_shape=jax.ShapeDtypeStruct((cp, e), w_pad.dtype),
    )(w_pad, b_row)


def _gather_rows(table, idx3, n_rows, chunk):
    _, e = table.shape
    nw, k, _ = idx3.shape
    mesh = plsc.VectorSubcoreMesh(core_axis_name="c", subcore_axis_name="s")
    rows_per_w = k * chunk

    @functools.partial(
        pl.kernel,
        out_type=jax.ShapeDtypeStruct((n_rows, e), jnp.float32),
        mesh=mesh,
        scratch_types=[
            pltpu.VMEM((k, chunk), jnp.int32),
            pltpu.VMEM((chunk, e), jnp.float32),
            pltpu.SemaphoreType.DMA,
        ],
    )
    def k_fn(table_hbm, idx_hbm, out_hbm, idx_v, rows_v, sem):
        wid = lax.axis_index("s") * 2 + lax.axis_index("c")
        base = wid * rows_per_w
        pltpu.sync_copy(idx_hbm.at[wid], idx_v)
        for c in range(k):
            pltpu.async_copy(table_hbm.at[idx_v.at[c]], rows_v, sem).wait()
            pltpu.sync_copy(rows_v, out_hbm.at[pl.ds(base + c * chunk, chunk)])

    return k_fn(table, idx3)


def kernel(x, W, b):
    e, c = W.shape
    cp = (c + _LANES - 1) // _LANES * _LANES
    w_pad = jnp.pad(W, ((0, 0), (0, cp - c)))
    table = _build_table(w_pad, b.reshape(1, e))

    n = x.size
    nw = 32
    chunk = 128
    k = n // (nw * chunk)
    idx3 = x.reshape(nw, k, chunk).astype(jnp.int32)
    out = _gather_rows(table, idx3, n, chunk)
    return out.reshape(*x.shape, e)


# speedup vs baseline: 1.0297x; 1.0297x over previous
import functools

import jax
import jax.numpy as jnp
from jax import lax
from jax.experimental import pallas as pl
from jax.experimental.pallas import tpu as pltpu
from jax.experimental.pallas import tpu_sc as plsc

_LANES = 128


def _table_body(w_ref, b_ref, out_ref):
    out_ref[...] = w_ref[...].T + b_ref[...]


def _build_table(w_pad, b_row):
    e, cp = w_pad.shape
    return pl.pallas_call(
        _table_body,
        out_shape=jax.ShapeDtypeStruct((cp, e), w_pad.dtype),
    )(w_pad, b_row)


def _gather_rows(table, idx3, n_rows, chunk):
    _, e = table.shape
    nw, k, _ = idx3.shape
    mesh = plsc.VectorSubcoreMesh(core_axis_name="c", subcore_axis_name="s")
    rows_per_w = k * chunk

    nbuf = 4
    look = 3

    @functools.partial(
        pl.kernel,
        out_type=jax.ShapeDtypeStruct((n_rows, e), jnp.float32),
        mesh=mesh,
        scratch_types=[
            pltpu.VMEM((k, chunk), jnp.int32),
            pltpu.VMEM((nbuf, chunk, e), jnp.float32),
            [pltpu.SemaphoreType.DMA] * nbuf,
            [pltpu.SemaphoreType.DMA] * nbuf,
        ],
    )
    def k_fn(table_hbm, idx_hbm, out_hbm, idx_v, rows_v, sems_g, sems_s):
        wid = lax.axis_index("s") * 2 + lax.axis_index("c")
        base = wid * rows_per_w
        pltpu.sync_copy(idx_hbm.at[wid], idx_v)

        gathers = [None] * k
        stores = [None] * k

        def start_gather(j):
            gathers[j] = pltpu.async_copy(
                table_hbm.at[idx_v.at[j]], rows_v.at[j % nbuf], sems_g[j % nbuf]
            )

        for j in range(min(look, k)):
            start_gather(j)
        for c in range(k):
            j = c + look
            if j < k:
                if j >= nbuf:
                    stores[j - nbuf].wait()
                start_gather(j)
            gathers[c].wait()
            stores[c] = pltpu.async_copy(
                rows_v.at[c % nbuf],
                out_hbm.at[pl.ds(base + c * chunk, chunk)],
                sems_s[c % nbuf],
            )
        for c in range(max(0, k - nbuf), k):
            if stores[c] is not None:
                stores[c].wait()

    return k_fn(table, idx3)


def kernel(x, W, b):
    e, c = W.shape
    cp = (c + _LANES - 1) // _LANES * _LANES
    w_pad = jnp.pad(W, ((0, 0), (0, cp - c)))
    table = _build_table(w_pad, b.reshape(1, e))

    n = x.size
    nw = 32
    chunk = 128
    k = n // (nw * chunk)
    idx3 = x.reshape(nw, k, chunk).astype(jnp.int32)
    out = _gather_rows(table, idx3, n, chunk)
    return out.reshape(*x.shape, e)


# speedup vs baseline: 1.5238x; 1.4799x over previous
import functools

import jax
import jax.numpy as jnp
from jax import lax
from jax.experimental import pallas as pl
from jax.experimental.pallas import tpu as pltpu
from jax.experimental.pallas import tpu_sc as plsc

_LANES = 128


def _table_body(w_ref, b_ref, out_ref):
    out_ref[...] = w_ref[...].T + b_ref[...]


def _build_table(w_pad, b_row):
    e, cp = w_pad.shape
    return pl.pallas_call(
        _table_body,
        out_shape=jax.ShapeDtypeStruct((cp, e), w_pad.dtype),
    )(w_pad, b_row)


def _gather_rows(table, idx3, n_i, t):
    _, e = table.shape
    nw, k, chunk = idx3.shape
    ipc = chunk // t
    i_per_tile = n_i // nw
    group = 4
    mesh = plsc.VectorSubcoreMesh(core_axis_name="c", subcore_axis_name="s")

    @functools.partial(
        pl.kernel,
        out_type=jax.ShapeDtypeStruct((n_i, t, e), jnp.float32),
        mesh=mesh,
        scratch_types=[
            pltpu.VMEM((k, chunk), jnp.int32),
            pltpu.VMEM((group, chunk, e), jnp.float32),
            [pltpu.SemaphoreType.DMA] * group,
            [pltpu.SemaphoreType.DMA] * group,
        ],
        compiler_params=pltpu.CompilerParams(use_tc_tiling_on_sc=True),
    )
    def k_fn(table_hbm, idx_hbm, out_hbm, idx_v, rows_v, sems_g, sems_s):
        wid = lax.axis_index("s") * 2 + lax.axis_index("c")
        i_base = wid * i_per_tile
        pltpu.sync_copy(idx_hbm.at[wid], idx_v)

        def body(g, carry):
            c0 = g * group
            gathers = [
                pltpu.async_copy(
                    table_hbm.at[idx_v.at[c0 + u]], rows_v.at[u], sems_g[u]
                )
                for u in range(group)
            ]
            stores = []
            for u in range(group):
                gathers[u].wait()
                i0 = i_base + (c0 + u) * ipc
                for b in range(ipc):
                    stores.append(
                        pltpu.async_copy(
                            rows_v.at[u, pl.ds(b * t, t)],
                            out_hbm.at[i0 + b],
                            sems_s[u],
                        )
                    )
            for s in stores:
                s.wait()
            return carry

        lax.fori_loop(0, k // group, body, 0)

    return k_fn(table, idx3)


def kernel(x, W, b):
    e, c = W.shape
    cp = (c + _LANES - 1) // _LANES * _LANES
    w_pad = jnp.pad(W, ((0, 0), (0, cp - c)))
    table = _build_table(w_pad, b.reshape(1, e))

    n_i, t = x.shape
    nw = 32
    ipc = 4
    chunk = ipc * t
    k = n_i // (nw * ipc)
    idx3 = x.reshape(nw, k, chunk).astype(jnp.int32)
    return _gather_rows(table, idx3, n_i, t)


# speedup vs baseline: 2.1124x; 1.3862x over previous
import functools

import jax
import jax.numpy as jnp
from jax import lax
from jax.experimental import pallas as pl
from jax.experimental.pallas import tpu as pltpu
from jax.experimental.pallas import tpu_sc as plsc

_LANES = 128


def _table_body(w_ref, b_ref, out_ref):
    out_ref[...] = w_ref[...].T + b_ref[...]


def _build_table(w_pad, b_row):
    e, cp = w_pad.shape
    return pl.pallas_call(
        _table_body,
        out_shape=jax.ShapeDtypeStruct((cp, e), w_pad.dtype),
    )(w_pad, b_row)


def _gather_rows(table, idx3, n_i, t):
    cp, e = table.shape
    nw, k, chunk = idx3.shape
    ipc = chunk // t
    i_per_tile = n_i // nw
    group = 4
    nbuf = 2 * group
    mesh = plsc.VectorSubcoreMesh(core_axis_name="c", subcore_axis_name="s")

    @functools.partial(
        pl.kernel,
        out_type=jax.ShapeDtypeStruct((n_i, t, e), jnp.float32),
        mesh=mesh,
        scratch_types=[
            pltpu.VMEM_SHARED((cp, e), jnp.float32),
            pltpu.VMEM((k, chunk), jnp.int32),
            pltpu.VMEM((nbuf, chunk, e), jnp.float32),
            [pltpu.SemaphoreType.DMA] * nbuf,
            [pltpu.SemaphoreType.DMA] * nbuf,
        ],
        compiler_params=pltpu.CompilerParams(use_tc_tiling_on_sc=True),
    )
    def k_fn(table_hbm, idx_hbm, out_hbm, table_sh, idx_v, rows_v, sems_g, sems_s):
        sid = lax.axis_index("s")
        wid = sid * 2 + lax.axis_index("c")
        i_base = wid * i_per_tile

        @pl.when(sid == 0)
        def _stage():
            pltpu.sync_copy(table_hbm, table_sh)

        pltpu.sync_copy(idx_hbm.at[wid], idx_v)
        plsc.subcore_barrier()

        def drain_stores(p):
            for u in range(group):
                for b in range(ipc):
                    pltpu.make_async_copy(
                        rows_v.at[p * group + u, pl.ds(b * t, t)],
                        out_hbm.at[i_base],
                        sems_s[p * group + u],
                    ).wait()

        def fire_gathers(g, p):
            c0 = g * 2 * group + p * group
            return [
                pltpu.async_copy(
                    table_sh.at[idx_v.at[c0 + u]],
                    rows_v.at[p * group + u],
                    sems_g[p * group + u],
                )
                for u in range(group)
            ]

        def fire_stores(g, p, gathers):
            c0 = g * 2 * group + p * group
            for u in range(group):
                gathers[u].wait()
                i0 = i_base + (c0 + u) * ipc
                for b in range(ipc):
                    pltpu.async_copy(
                        rows_v.at[p * group + u, pl.ds(b * t, t)],
                        out_hbm.at[i0 + b],
                        sems_s[p * group + u],
                    )

        def body(g, carry):
            @pl.when(g > 0)
            def _():
                drain_stores(0)

            ga = fire_gathers(g, 0)

            @pl.when(g > 0)
            def _():
                drain_stores(1)

            gb = fire_gathers(g, 1)
            fire_stores(g, 0, ga)
            fire_stores(g, 1, gb)
            return carry

        lax.fori_loop(0, k // (2 * group), body, 0)
        drain_stores(0)
        drain_stores(1)

    return k_fn(table, idx3)


def kernel(x, W, b):
    e, c = W.shape
    cp = (c + _LANES - 1) // _LANES * _LANES
    w_pad = jnp.pad(W, ((0, 0), (0, cp - c)))
    table = _build_table(w_pad, b.reshape(1, e))

    n_i, t = x.shape
    nw = 32
    ipc = 4
    chunk = ipc * t
    k = n_i // (nw * ipc)
    idx3 = x.reshape(nw, k, chunk).astype(jnp.int32)
    return _gather_rows(table, idx3, n_i, t)
